# baseline (speedup 1.0000x reference)
import jax
import jax.numpy as jnp
from jax import lax
from jax.experimental import pallas as pl
from jax.experimental.pallas import tpu as pltpu
from jax.experimental.pallas import tpu_sc as plsc

_HW = 112 * 112
_K = int(0.5 * _HW)
_PLANES = 4 * 192
_NW = 32
_PPW = _PLANES // _NW
_NV = _HW // 16


def _sc_body(x_hbm, o_hbm, buf, cand, hist):
    wid = lax.axis_index("s") * 2 + lax.axis_index("c")
    lane_iota = lax.iota(jnp.int32, 16)
    lanes256 = lane_iota * 256
    ones = jnp.ones((16,), jnp.int32)
    zeros16 = jnp.zeros((16,), jnp.int32)

    def per_plane(p_local, _):
        plane = wid * _PPW + p_local
        pltpu.sync_copy(x_hbm.at[plane], buf)

        @plsc.parallel_loop(0, 256, unroll=8)
        def zero_it(j):
            hist[pl.ds(j * 16, 16)] = zeros16

        @plsc.parallel_loop(0, _NV, unroll=8)
        def hist_it(i):
            v = buf[pl.ds(i * 16, 16)]
            b = (lax.bitcast_convert_type(v, jnp.int32) & 0x7FFFFFFF) >> 23
            plsc.addupdate_scatter(hist, [lanes256 + b], ones)

        def scan_it(cc, carry):
            suffix, cnt_v, snext_v = carry
            c = 15 - cc
            t = [hist[pl.ds(l * 256 + c * 16, 16)] for l in range(16)]
            for stride in (8, 4, 2, 1):
                t = [t[j] + t[j + stride] for j in range(stride)]
            s = lax.rev(plsc.cumsum(lax.rev(t[0], (0,))), (0,)) + suffix
            suffix = jnp.max(s)
            cnt_v = cnt_v + jnp.where(s >= _K, 1, 0)
            snext_v = jnp.maximum(snext_v, jnp.where(s < _K, s, 0))
            return suffix, cnt_v, snext_v

        _, cnt_v, snext_v = lax.fori_loop(
            0, 16, scan_it, (jnp.int32(0), zeros16, zeros16))
        b1 = jnp.sum(cnt_v) - 1
        k_rem = jnp.int32(_K) - jnp.max(snext_v)

        @plsc.parallel_loop(0, _NV, unroll=8, carry=zeros16)
        def comp_it(i, off_v):
            v = buf[pl.ds(i * 16, 16)]
            b = (lax.bitcast_convert_type(v, jnp.int32) & 0x7FFFFFFF) >> 23
            m = b == b1
            mi = jnp.where(m, 1, 0)
            pos = plsc.cumsum(mi) - mi + off_v
            plsc.store_scatter(cand, [pos], v, mask=m)
            return off_v + plsc.all_reduce_population_count(m)

        n_cand = jnp.max(comp_it)
        nv2 = (n_cand + 15) >> 4

        def bs_it(_, carry):
            lo, hi = carry
            mid = lo + ((hi - lo + 1) >> 1)

            def cnt_it(i, acc_v):
                v = cand[pl.ds(i * 16, 16)]
                ab = lax.bitcast_convert_type(v, jnp.int32) & 0x7FFFFF
                valid = (lane_iota + i * 16) < n_cand
                return acc_v + jnp.where(valid & (ab >= mid), 1, 0)

            cnt = jnp.sum(lax.fori_loop(0, nv2, cnt_it, zeros16))
            ok = cnt >= k_rem
            return jnp.where(ok, mid, lo), jnp.where(ok, hi, mid - 1)

        lo23, _ = lax.fori_loop(0, 23, bs_it,
                                (jnp.int32(0), jnp.int32(0x7FFFFF)))
        thr = (b1 << 23) | lo23

        @plsc.parallel_loop(0, _NV, unroll=8)
        def mask_it(i):
            v = buf[pl.ds(i * 16, 16)]
            ab = lax.bitcast_convert_type(v, jnp.int32) & 0x7FFFFFFF
            buf[pl.ds(i * 16, 16)] = jnp.where(ab >= thr, v, 0.0)

        pltpu.sync_copy(buf, o_hbm.at[plane])
        return 0

    lax.fori_loop(0, _PPW, per_plane, 0)


@jax.jit
def _sc_call(x2):
    return pl.kernel(
        _sc_body,
        out_type=jax.ShapeDtypeStruct((_PLANES, _HW), jnp.float32),
        mesh=plsc.VectorSubcoreMesh(core_axis_name="c", subcore_axis_name="s"),
        compiler_params=pltpu.CompilerParams(needs_layout_passes=False),
        scratch_types=[
            pltpu.VMEM((_HW,), jnp.float32),
            pltpu.VMEM((_HW + 16,), jnp.float32),
            pltpu.VMEM((16 * 256,), jnp.int32),
        ],
    )(x2)


def kernel(x):
    B, C, H, W = x.shape
    x2 = x.reshape(B * C, H * W)
    return _sc_call(x2).reshape(B, C, H, W)
